# baseline (speedup 1.0000x reference)
import jax
import jax.numpy as jnp
from jax import lax
from jax.experimental import pallas as pl
from jax.experimental.pallas import tpu as pltpu
from jax.experimental.pallas import tpu_sc as plsc

_EMBED = 16
_NC = 2
_NS = 16
_NW = _NC * _NS
_CHUNK = 128
_TW = 8192
_TBLK = 123


def _tpose(x_ref, o_ref):
    o_ref[...] = jax.lax.dot_general(
        x_ref[...], jnp.eye(_EMBED, dtype=jnp.float32),
        dimension_numbers=(((0,), (0,)), ((), ())),
        preferred_element_type=jnp.float32)


def _repack(table_t):
    return pl.pallas_call(
        _tpose,
        grid=(_TBLK,),
        in_specs=[pl.BlockSpec((16, _TW), lambda t: (0, t))],
        out_specs=pl.BlockSpec((_TW, 16), lambda t: (t, 0)),
        out_shape=jax.ShapeDtypeStruct((_TW * _TBLK, 16), jnp.float32),
    )(table_t)


def _sc_gather_dot(uep, bep, ub8, bb8, uidx, bidx,
                   partials, ubg, bbg,
                   idxu, idxb, qeu, qeb, qbu, qbb, urows, brows, ub8r, bb8r,
                   ubv, bbv, accv, sem_rows, sem_bias):
    batch = uidx.shape[0]
    bpw = batch // _NW
    nchunk = bpw // _CHUNK
    wid = lax.axis_index("s") * _NC + lax.axis_index("c")
    base = wid * bpw

    pltpu.sync_copy(uidx.at[pl.ds(base, bpw)], idxu)
    pltpu.sync_copy(bidx.at[pl.ds(base, bpw)], idxb)

    def qbody(j, carry):
        sl = pl.ds(j * 16, 16)
        qeu[sl] = lax.shift_right_logical(idxu[sl], 3)
        qeb[sl] = lax.shift_right_logical(idxb[sl], 3)
        return carry

    lax.fori_loop(0, bpw // 16, qbody, 0)

    copies = []
    for j in range(nchunk):
        sl = pl.ds(j * _CHUNK, _CHUNK)
        copies.append(pltpu.async_copy(uep.at[idxu.at[sl]], urows.at[sl], sem_rows))
        copies.append(pltpu.async_copy(bep.at[idxb.at[sl]], brows.at[sl], sem_rows))
        copies.append(pltpu.async_copy(ub8.at[qeu.at[sl]], ub8r.at[sl], sem_bias))
        copies.append(pltpu.async_copy(bb8.at[qeb.at[sl]], bb8r.at[sl], sem_bias))
    for c in copies:
        c.wait()

    def dot_body(i, acc):
        return acc + urows[i] * brows[i]

    acc = lax.fori_loop(0, bpw, dot_body, jnp.zeros((_EMBED,), jnp.float32))
    accv[...] = acc
    pltpu.sync_copy(accv, partials.at[wid])

    def bias_sel(j, carry):
        sl = pl.ds(j * 16, 16)
        rows = jax.lax.iota(jnp.int32, 16) + j * 16
        ubv[sl] = plsc.load_gather(ub8r, [rows, idxu[sl] & 7])
        bbv[sl] = plsc.load_gather(bb8r, [rows, idxb[sl] & 7])
        return carry

    lax.fori_loop(0, bpw // 16, bias_sel, 0)

    pltpu.sync_copy(ubv, ubg.at[wid])
    pltpu.sync_copy(bbv, bbg.at[wid])


def _finish(p_ref, ub_ref, bb_ref, o_ref):
    s = jnp.sum(p_ref[...])
    x = s + ub_ref[...] + bb_ref[...]
    o_ref[...] = 1.0 / (1.0 + jnp.exp(-x))


def kernel(inputs, user_emb, user_bias, book_emb, book_bias):
    batch = inputs.shape[0]
    bpw = batch // _NW
    uidx = inputs[:, 0]
    bidx = inputs[:, 1]
    uep = _repack(user_emb.T)
    bep = _repack(book_emb.T)
    ub8 = user_bias.reshape(-1, 8)
    bb8 = book_bias.reshape(-1, 8)

    mesh = plsc.VectorSubcoreMesh(core_axis_name="c", subcore_axis_name="s")
    sc = pl.kernel(
        _sc_gather_dot,
        mesh=mesh,
        compiler_params=pltpu.CompilerParams(
            use_tc_tiling_on_sc=False, needs_layout_passes=False),
        out_type=[
            jax.ShapeDtypeStruct((_NW, _EMBED), jnp.float32),
            jax.ShapeDtypeStruct((_NW, bpw), jnp.float32),
            jax.ShapeDtypeStruct((_NW, bpw), jnp.float32),
        ],
        scratch_types=[
            pltpu.VMEM((bpw,), jnp.int32),
            pltpu.VMEM((bpw,), jnp.int32),
            pltpu.VMEM((bpw,), jnp.int32),
            pltpu.VMEM((bpw,), jnp.int32),
            pltpu.VMEM((bpw,), jnp.int32),
            pltpu.VMEM((bpw,), jnp.int32),
            pltpu.VMEM((bpw, _EMBED), jnp.float32),
            pltpu.VMEM((bpw, _EMBED), jnp.float32),
            pltpu.VMEM((bpw, 8), jnp.float32),
            pltpu.VMEM((bpw, 8), jnp.float32),
            pltpu.VMEM((bpw,), jnp.float32),
            pltpu.VMEM((bpw,), jnp.float32),
            pltpu.VMEM((_EMBED,), jnp.float32),
            pltpu.SemaphoreType.DMA,
            pltpu.SemaphoreType.DMA,
        ],
    )
    partials, ubg, bbg = sc(uep, bep, ub8, bb8, uidx, bidx)

    out = pl.pallas_call(
        _finish,
        out_shape=jax.ShapeDtypeStruct((_NW, bpw), jnp.float32),
    )(partials, ubg, bbg)
    return out.reshape(batch, 1)


# speedup vs baseline: 1.0579x; 1.0579x over previous
import jax
import jax.numpy as jnp
from jax import lax
from jax.experimental import pallas as pl
from jax.experimental.pallas import tpu as pltpu
from jax.experimental.pallas import tpu_sc as plsc

_EMBED = 16
_NC = 2
_NS = 16
_NW = _NC * _NS
_CHUNK = 128
_TW = 16384
_TBLK = 62


def _tpose(x_ref, o_ref):
    o_ref[...] = jax.lax.dot_general(
        x_ref[...], jnp.eye(_EMBED, dtype=jnp.float32),
        dimension_numbers=(((0,), (0,)), ((), ())),
        preferred_element_type=jnp.float32)


def _repack(table_t):
    return pl.pallas_call(
        _tpose,
        grid=(_TBLK,),
        in_specs=[pl.BlockSpec((16, _TW), lambda t: (0, t))],
        out_specs=pl.BlockSpec((_TW, 16), lambda t: (t, 0)),
        out_shape=jax.ShapeDtypeStruct((_TW * _TBLK, 16), jnp.float32),
    )(table_t)


def _sc_gather_dot(uep, bep, ub8, bb8, uidx, bidx,
                   partials, ubg, bbg,
                   idxu, idxb, qeu, qeb, qbu, qbb, urows, brows, ub8r, bb8r,
                   ubv, bbv, accv, sem_rows, sem_bias):
    batch = uidx.shape[0]
    bpw = batch // _NW
    nchunk = bpw // _CHUNK
    wid = lax.axis_index("s") * _NC + lax.axis_index("c")
    base = wid * bpw

    pltpu.sync_copy(uidx.at[pl.ds(base, bpw)], idxu)
    pltpu.sync_copy(bidx.at[pl.ds(base, bpw)], idxb)

    def qbody(j, carry):
        sl = pl.ds(j * 16, 16)
        qeu[sl] = lax.shift_right_logical(idxu[sl], 3)
        qeb[sl] = lax.shift_right_logical(idxb[sl], 3)
        return carry

    lax.fori_loop(0, bpw // 16, qbody, 0)

    copies = []
    for j in range(nchunk):
        sl = pl.ds(j * _CHUNK, _CHUNK)
        copies.append(pltpu.async_copy(uep.at[idxu.at[sl]], urows.at[sl], sem_rows))
        copies.append(pltpu.async_copy(bep.at[idxb.at[sl]], brows.at[sl], sem_rows))
        copies.append(pltpu.async_copy(ub8.at[qeu.at[sl]], ub8r.at[sl], sem_bias))
        copies.append(pltpu.async_copy(bb8.at[qeb.at[sl]], bb8r.at[sl], sem_bias))
    for c in copies:
        c.wait()

    def dot_body(i, acc):
        return acc + urows[i] * brows[i]

    acc = lax.fori_loop(0, bpw, dot_body, jnp.zeros((_EMBED,), jnp.float32))
    accv[...] = acc
    pltpu.sync_copy(accv, partials.at[wid])

    def bias_sel(j, carry):
        sl = pl.ds(j * 16, 16)
        rows = jax.lax.iota(jnp.int32, 16) + j * 16
        ubv[sl] = plsc.load_gather(ub8r, [rows, idxu[sl] & 7])
        bbv[sl] = plsc.load_gather(bb8r, [rows, idxb[sl] & 7])
        return carry

    lax.fori_loop(0, bpw // 16, bias_sel, 0)

    pltpu.sync_copy(ubv, ubg.at[wid])
    pltpu.sync_copy(bbv, bbg.at[wid])


def _finish(p_ref, ub_ref, bb_ref, o_ref):
    s = jnp.sum(p_ref[...])
    x = s + ub_ref[...] + bb_ref[...]
    o_ref[...] = 1.0 / (1.0 + jnp.exp(-x))


def kernel(inputs, user_emb, user_bias, book_emb, book_bias):
    batch = inputs.shape[0]
    bpw = batch // _NW
    uidx = inputs[:, 0]
    bidx = inputs[:, 1]
    uep = _repack(user_emb.T)
    bep = _repack(book_emb.T)
    ub8 = user_bias.reshape(-1, 8)
    bb8 = book_bias.reshape(-1, 8)

    mesh = plsc.VectorSubcoreMesh(core_axis_name="c", subcore_axis_name="s")
    sc = pl.kernel(
        _sc_gather_dot,
        mesh=mesh,
        compiler_params=pltpu.CompilerParams(
            use_tc_tiling_on_sc=False, needs_layout_passes=False),
        out_type=[
            jax.ShapeDtypeStruct((_NW, _EMBED), jnp.float32),
            jax.ShapeDtypeStruct((_NW, bpw), jnp.float32),
            jax.ShapeDtypeStruct((_NW, bpw), jnp.float32),
        ],
        scratch_types=[
            pltpu.VMEM((bpw,), jnp.int32),
            pltpu.VMEM((bpw,), jnp.int32),
            pltpu.VMEM((bpw,), jnp.int32),
            pltpu.VMEM((bpw,), jnp.int32),
            pltpu.VMEM((bpw,), jnp.int32),
            pltpu.VMEM((bpw,), jnp.int32),
            pltpu.VMEM((bpw, _EMBED), jnp.float32),
            pltpu.VMEM((bpw, _EMBED), jnp.float32),
            pltpu.VMEM((bpw, 8), jnp.float32),
            pltpu.VMEM((bpw, 8), jnp.float32),
            pltpu.VMEM((bpw,), jnp.float32),
            pltpu.VMEM((bpw,), jnp.float32),
            pltpu.VMEM((_EMBED,), jnp.float32),
            pltpu.SemaphoreType.DMA,
            pltpu.SemaphoreType.DMA,
        ],
    )
    partials, ubg, bbg = sc(uep, bep, ub8, bb8, uidx, bidx)

    out = pl.pallas_call(
        _finish,
        out_shape=jax.ShapeDtypeStruct((_NW, bpw), jnp.float32),
    )(partials, ubg, bbg)
    return out.reshape(batch, 1)


# speedup vs baseline: 1.4571x; 1.3774x over previous
import jax
import jax.numpy as jnp
from jax import lax
from jax.experimental import pallas as pl
from jax.experimental.pallas import tpu as pltpu
from jax.experimental.pallas import tpu_sc as plsc

_EMBED = 16
_NC = 2
_NS = 16
_NW = _NC * _NS
_CHUNK = 128


def _sc_gather_dot(user_emb, book_emb, ub8, bb8, uidx, bidx,
                   partials, ubg, bbg,
                   idxu, idxb, qidxu, qidxb, urows, brows, ub8r, bb8r,
                   ubv, bbv, accv, sem_rows, sem_bias):
    batch = uidx.shape[0]
    bpw = batch // _NW
    nchunk = bpw // _CHUNK
    wid = lax.axis_index("s") * _NC + lax.axis_index("c")
    base = wid * bpw

    pltpu.sync_copy(uidx.at[pl.ds(base, bpw)], idxu)
    pltpu.sync_copy(bidx.at[pl.ds(base, bpw)], idxb)

    def qbody(j, carry):
        sl = pl.ds(j * 16, 16)
        qidxu[sl] = lax.shift_right_logical(idxu[sl], 3)
        qidxb[sl] = lax.shift_right_logical(idxb[sl], 3)
        return carry

    lax.fori_loop(0, bpw // 16, qbody, 0)

    copies = []
    for j in range(nchunk):
        sl = pl.ds(j * _CHUNK, _CHUNK)
        copies.append(pltpu.async_copy(user_emb.at[idxu.at[sl]], urows.at[sl], sem_rows))
        copies.append(pltpu.async_copy(book_emb.at[idxb.at[sl]], brows.at[sl], sem_rows))
        copies.append(pltpu.async_copy(ub8.at[qidxu.at[sl]], ub8r.at[sl], sem_bias))
        copies.append(pltpu.async_copy(bb8.at[qidxb.at[sl]], bb8r.at[sl], sem_bias))
    for c in copies:
        c.wait()

    def bias_sel(j, carry):
        sl = pl.ds(j * 16, 16)
        rows = jax.lax.iota(jnp.int32, 16) + j * 16
        ubv[sl] = plsc.load_gather(ub8r, [rows, idxu[sl] & 7])
        bbv[sl] = plsc.load_gather(bb8r, [rows, idxb[sl] & 7])
        return carry

    lax.fori_loop(0, bpw // 16, bias_sel, 0)

    pltpu.sync_copy(ubv, ubg.at[wid])
    pltpu.sync_copy(bbv, bbg.at[wid])

    def dot_body(i, acc):
        return acc + urows[i] * brows[i]

    acc = lax.fori_loop(0, bpw, dot_body, jnp.zeros((_EMBED,), jnp.float32))
    accv[...] = acc
    pltpu.sync_copy(accv, partials.at[wid])


def _finish(p_ref, ub_ref, bb_ref, o_ref):
    s = jnp.sum(p_ref[...])
    x = s + ub_ref[...] + bb_ref[...]
    o_ref[...] = 1.0 / (1.0 + jnp.exp(-x))


def kernel(inputs, user_emb, user_bias, book_emb, book_bias):
    batch = inputs.shape[0]
    bpw = batch // _NW
    uidx = inputs[:, 0]
    bidx = inputs[:, 1]
    ub8 = user_bias.reshape(-1, 8)
    bb8 = book_bias.reshape(-1, 8)

    mesh = plsc.VectorSubcoreMesh(core_axis_name="c", subcore_axis_name="s")
    sc = pl.kernel(
        _sc_gather_dot,
        mesh=mesh,
        compiler_params=pltpu.CompilerParams(
            use_tc_tiling_on_sc=False, needs_layout_passes=False),
        out_type=[
            jax.ShapeDtypeStruct((_NW, _EMBED), jnp.float32),
            jax.ShapeDtypeStruct((_NW, bpw), jnp.float32),
            jax.ShapeDtypeStruct((_NW, bpw), jnp.float32),
        ],
        scratch_types=[
            pltpu.VMEM((bpw,), jnp.int32),
            pltpu.VMEM((bpw,), jnp.int32),
            pltpu.VMEM((bpw,), jnp.int32),
            pltpu.VMEM((bpw,), jnp.int32),
            pltpu.VMEM((bpw, _EMBED), jnp.float32),
            pltpu.VMEM((bpw, _EMBED), jnp.float32),
            pltpu.VMEM((bpw, 8), jnp.float32),
            pltpu.VMEM((bpw, 8), jnp.float32),
            pltpu.VMEM((bpw,), jnp.float32),
            pltpu.VMEM((bpw,), jnp.float32),
            pltpu.VMEM((_EMBED,), jnp.float32),
            pltpu.SemaphoreType.DMA,
            pltpu.SemaphoreType.DMA,
        ],
    )
    partials, ubg, bbg = sc(user_emb, book_emb, ub8, bb8, uidx, bidx)

    out = pl.pallas_call(
        _finish,
        out_shape=jax.ShapeDtypeStruct((_NW, bpw), jnp.float32),
    )(partials, ubg, bbg)
    return out.reshape(batch, 1)
